# Initial kernel scaffold; baseline (speedup 1.0000x reference)
import jax
import jax.numpy as jnp
from jax.experimental import pallas as pl


def kernel(x, edge_index, W1_l, W1_r, b1, W2_l, W2_r, b2, Wfc, bfc):
    raise NotImplementedError("write your pallas kernel here")


# speedup vs baseline: 3.1453x; 3.1453x over previous
import functools

import jax
import jax.numpy as jnp
from jax import lax
from jax.experimental import pallas as pl
from jax.experimental.pallas import tpu as pltpu
from jax.experimental.pallas import tpu_sc as plsc

N = 10000
D = 128
NC, NS = 2, 16
NW = NC * NS
LANES = 16
CHUNK = 128
ACC_ROWS = 10240
ZROWS = ACC_ROWS // NS


def _build_sc_aggregate(k_per_worker: int, with_cnt: bool):
    mesh = plsc.VectorSubcoreMesh(
        core_axis_name="c", subcore_axis_name="s", num_cores=NC, num_subcores=NS
    )
    out_type = [jax.ShapeDtypeStruct((NC, ACC_ROWS, D), jnp.float32)]
    scratch = [
        pltpu.VMEM((k_per_worker, CHUNK), jnp.int32),
        pltpu.VMEM((k_per_worker, CHUNK), jnp.int32),
        pltpu.VMEM((CHUNK, D), jnp.float32),
        pltpu.VMEM_SHARED((ACC_ROWS, D), jnp.float32),
        pltpu.SemaphoreType.DMA,
    ]
    if with_cnt:
        out_type.append(jax.ShapeDtypeStruct((NC, ACC_ROWS, LANES), jnp.float32))
        scratch += [
            pltpu.VMEM((CHUNK, LANES), jnp.float32),
            pltpu.VMEM_SHARED((ACC_ROWS, LANES), jnp.float32),
        ]

    def body(*refs):
        if with_cnt:
            (feats, src_h, dst_h, zacc_h, zcnt_h, ones_h,
             out_sum, out_cnt, src_v, dst_v, rows_v, acc, sem,
             ones_v, cnt_acc) = refs
        else:
            (feats, src_h, dst_h, zacc_h,
             out_sum, src_v, dst_v, rows_v, acc, sem) = refs
        cid = lax.axis_index("c")
        sid = lax.axis_index("s")
        wid = cid * NS + sid

        pltpu.sync_copy(zacc_h, acc.at[pl.ds(sid * ZROWS, ZROWS)])
        if with_cnt:
            pltpu.sync_copy(zcnt_h, cnt_acc.at[pl.ds(sid * ZROWS, ZROWS)])
            pltpu.sync_copy(ones_h, ones_v)
        pltpu.sync_copy(src_h.at[pl.ds(wid * k_per_worker, k_per_worker)], src_v)
        pltpu.sync_copy(dst_h.at[pl.ds(wid * k_per_worker, k_per_worker)], dst_v)
        plsc.subcore_barrier()

        def step(j, carry):
            pltpu.async_copy(feats.at[src_v.at[j]], rows_v, sem).wait()
            pltpu.sync_copy(rows_v, acc.at[dst_v.at[j]], add=True)
            if with_cnt:
                pltpu.sync_copy(ones_v, cnt_acc.at[dst_v.at[j]], add=True)
            return carry

        lax.fori_loop(0, k_per_worker, step, 0)
        plsc.subcore_barrier()

        pltpu.sync_copy(
            acc.at[pl.ds(sid * ZROWS, ZROWS)],
            out_sum.at[cid, pl.ds(sid * ZROWS, ZROWS)],
        )
        if with_cnt:
            pltpu.sync_copy(
                cnt_acc.at[pl.ds(sid * ZROWS, ZROWS)],
                out_cnt.at[cid, pl.ds(sid * ZROWS, ZROWS)],
            )

    return pl.kernel(
        body, out_type=out_type, mesh=mesh, scratch_types=scratch,
        compiler_params=pltpu.CompilerParams(use_tc_tiling_on_sc=False),
    )


def _tc_layer(S, C, xin, W_l, W_r, b, Wfc=None, bfc=None):
    BR = 1000
    final = Wfc is not None

    def body(*refs):
        if final:
            S_r, C_r, x_r, Wl_r, Wr_r, b_r, Wfc_r, bfc_r, o_r = refs
        else:
            S_r, C_r, x_r, Wl_r, Wr_r, b_r, o_r = refs
        s = S_r[0] + S_r[1]
        cnt = C_r[0, :, :1] + C_r[1, :, :1]
        aggr = s / jnp.maximum(cnt, 1.0)
        h = (jnp.dot(aggr, Wl_r[...], preferred_element_type=jnp.float32)
             + jnp.dot(x_r[...], Wr_r[...], preferred_element_type=jnp.float32)
             + b_r[...])
        h = jnp.maximum(h, 0.0)
        if final:
            o_r[...] = (jnp.dot(h, Wfc_r[...], preferred_element_type=jnp.float32)
                        + bfc_r[...])
        else:
            o_r[...] = h

    in_specs = [
        pl.BlockSpec((NC, BR, D), lambda i: (0, i, 0)),
        pl.BlockSpec((NC, BR, LANES), lambda i: (0, i, 0)),
        pl.BlockSpec((BR, D), lambda i: (i, 0)),
        pl.BlockSpec((D, D), lambda i: (0, 0)),
        pl.BlockSpec((D, D), lambda i: (0, 0)),
        pl.BlockSpec((1, D), lambda i: (0, 0)),
    ]
    args = [S, C, xin, W_l, W_r, b.reshape(1, D)]
    if final:
        in_specs += [pl.BlockSpec((D, 1), lambda i: (0, 0)),
                     pl.BlockSpec((1, 1), lambda i: (0, 0))]
        args += [Wfc, bfc.reshape(1, 1)]
        out_spec = pl.BlockSpec((BR, 1), lambda i: (i, 0))
        out_shape = jax.ShapeDtypeStruct((N, 1), jnp.float32)
    else:
        out_spec = pl.BlockSpec((BR, D), lambda i: (i, 0))
        out_shape = jax.ShapeDtypeStruct((N, D), jnp.float32)

    return pl.pallas_call(
        body, grid=(N // BR,), in_specs=in_specs, out_specs=out_spec,
        out_shape=out_shape,
    )(*args)


def kernel(x, edge_index, W1_l, W1_r, b1, W2_l, W2_r, b2, Wfc, bfc):
    e = edge_index.shape[1]
    quantum = NW * CHUNK * 8
    e_pad = ((e + quantum - 1) // quantum) * quantum
    k_per_worker = e_pad // (NW * CHUNK)
    pad = e_pad - e

    src = jnp.concatenate(
        [edge_index[0].astype(jnp.int32), jnp.zeros((pad,), jnp.int32)]
    ).reshape(-1, CHUNK)
    dst = jnp.concatenate(
        [edge_index[1].astype(jnp.int32), jnp.full((pad,), N, jnp.int32)]
    ).reshape(-1, CHUNK)
    zacc = jnp.zeros((ZROWS, D), jnp.float32)
    zcnt = jnp.zeros((ZROWS, LANES), jnp.float32)
    ones = jnp.ones((CHUNK, LANES), jnp.float32)

    agg1 = _build_sc_aggregate(k_per_worker, with_cnt=True)
    agg2 = _build_sc_aggregate(k_per_worker, with_cnt=False)

    S1, C = agg1(x, src, dst, zacc, zcnt, ones)
    h1 = _tc_layer(S1, C, x, W1_l, W1_r, b1)
    (S2,) = agg2(h1, src, dst, zacc)
    out = _tc_layer(S2, C, h1, W2_l, W2_r, b2, Wfc=Wfc, bfc=bfc)
    return out[:, 0]


# speedup vs baseline: 3.5633x; 1.1329x over previous
import functools

import jax
import jax.numpy as jnp
from jax import lax
from jax.experimental import pallas as pl
from jax.experimental.pallas import tpu as pltpu
from jax.experimental.pallas import tpu_sc as plsc

N = 10000
D = 128
NC, NS = 2, 16
NW = NC * NS
LANES = 16
CHUNK = 64
ACC_ROWS = 10112
ZROWS = ACC_ROWS // NS


def _build_sc_aggregate(k_per_worker: int, with_cnt: bool):
    mesh = plsc.VectorSubcoreMesh(
        core_axis_name="c", subcore_axis_name="s", num_cores=NC, num_subcores=NS
    )
    out_type = [jax.ShapeDtypeStruct((NC, ACC_ROWS, D), jnp.float32)]
    scratch = [
        pltpu.VMEM((k_per_worker, CHUNK), jnp.int32),
        pltpu.VMEM((k_per_worker, CHUNK), jnp.int32),
        pltpu.VMEM((CHUNK, D), jnp.float32),
        pltpu.VMEM((CHUNK, D), jnp.float32),
        pltpu.VMEM_SHARED((ACC_ROWS, D), jnp.float32),
        pltpu.SemaphoreType.DMA,
        pltpu.SemaphoreType.DMA,
    ]
    if with_cnt:
        out_type.append(jax.ShapeDtypeStruct((NC, ACC_ROWS, LANES), jnp.float32))
        scratch += [
            pltpu.VMEM((CHUNK, LANES), jnp.float32),
            pltpu.VMEM_SHARED((ACC_ROWS, LANES), jnp.float32),
        ]

    def body(*refs):
        if with_cnt:
            (feats, src_h, dst_h, zacc_h, zcnt_h, ones_h,
             out_sum, out_cnt, src_v, dst_v, rows_a, rows_b, acc, sem_a,
             sem_b, ones_v, cnt_acc) = refs
        else:
            (feats, src_h, dst_h, zacc_h,
             out_sum, src_v, dst_v, rows_a, rows_b, acc, sem_a, sem_b) = refs
        cid = lax.axis_index("c")
        sid = lax.axis_index("s")
        wid = cid * NS + sid

        pltpu.sync_copy(zacc_h, acc.at[pl.ds(sid * ZROWS, ZROWS)])
        if with_cnt:
            pltpu.sync_copy(zcnt_h, cnt_acc.at[pl.ds(sid * ZROWS, ZROWS)])
            pltpu.sync_copy(ones_h, ones_v)
        pltpu.sync_copy(src_h.at[pl.ds(wid * k_per_worker, k_per_worker)], src_v)
        pltpu.sync_copy(dst_h.at[pl.ds(wid * k_per_worker, k_per_worker)], dst_v)
        plsc.subcore_barrier()

        half = k_per_worker // 2

        def start(j, rows, sem):
            return pltpu.async_copy(feats.at[src_v.at[j]], rows, sem)

        def drain(j, rows, sem):
            pltpu.make_async_copy(feats.at[src_v.at[j]], rows, sem).wait()
            pltpu.sync_copy(rows, acc.at[dst_v.at[j]], add=True)
            if with_cnt:
                pltpu.sync_copy(ones_v, cnt_acc.at[dst_v.at[j]], add=True)

        start(0, rows_a, sem_a)

        def step(g, carry):
            j0 = 2 * g
            start(j0 + 1, rows_b, sem_b)
            drain(j0, rows_a, sem_a)
            start(jnp.minimum(j0 + 2, k_per_worker - 1), rows_a, sem_a)
            drain(j0 + 1, rows_b, sem_b)
            return carry

        lax.fori_loop(0, half, step, 0)
        pltpu.make_async_copy(
            feats.at[src_v.at[k_per_worker - 1]], rows_a, sem_a
        ).wait()
        plsc.subcore_barrier()

        pltpu.sync_copy(
            acc.at[pl.ds(sid * ZROWS, ZROWS)],
            out_sum.at[cid, pl.ds(sid * ZROWS, ZROWS)],
        )
        if with_cnt:
            pltpu.sync_copy(
                cnt_acc.at[pl.ds(sid * ZROWS, ZROWS)],
                out_cnt.at[cid, pl.ds(sid * ZROWS, ZROWS)],
            )

    return pl.kernel(
        body, out_type=out_type, mesh=mesh, scratch_types=scratch,
        compiler_params=pltpu.CompilerParams(use_tc_tiling_on_sc=False),
    )


def _tc_layer(S, C, xin, W_l, W_r, b, Wfc=None, bfc=None):
    BR = 1000
    final = Wfc is not None

    def body(*refs):
        if final:
            S_r, C_r, x_r, Wl_r, Wr_r, b_r, Wfc_r, bfc_r, o_r = refs
        else:
            S_r, C_r, x_r, Wl_r, Wr_r, b_r, o_r = refs
        s = S_r[0] + S_r[1]
        cnt = C_r[0, :, :1] + C_r[1, :, :1]
        aggr = s / jnp.maximum(cnt, 1.0)
        h = (jnp.dot(aggr, Wl_r[...], preferred_element_type=jnp.float32)
             + jnp.dot(x_r[...], Wr_r[...], preferred_element_type=jnp.float32)
             + b_r[...])
        h = jnp.maximum(h, 0.0)
        if final:
            o_r[...] = (jnp.dot(h, Wfc_r[...], preferred_element_type=jnp.float32)
                        + bfc_r[...])
        else:
            o_r[...] = h

    in_specs = [
        pl.BlockSpec((NC, BR, D), lambda i: (0, i, 0)),
        pl.BlockSpec((NC, BR, LANES), lambda i: (0, i, 0)),
        pl.BlockSpec((BR, D), lambda i: (i, 0)),
        pl.BlockSpec((D, D), lambda i: (0, 0)),
        pl.BlockSpec((D, D), lambda i: (0, 0)),
        pl.BlockSpec((1, D), lambda i: (0, 0)),
    ]
    args = [S, C, xin, W_l, W_r, b.reshape(1, D)]
    if final:
        in_specs += [pl.BlockSpec((D, 1), lambda i: (0, 0)),
                     pl.BlockSpec((1, 1), lambda i: (0, 0))]
        args += [Wfc, bfc.reshape(1, 1)]
        out_spec = pl.BlockSpec((BR, 1), lambda i: (i, 0))
        out_shape = jax.ShapeDtypeStruct((N, 1), jnp.float32)
    else:
        out_spec = pl.BlockSpec((BR, D), lambda i: (i, 0))
        out_shape = jax.ShapeDtypeStruct((N, D), jnp.float32)

    return pl.pallas_call(
        body, grid=(N // BR,), in_specs=in_specs, out_specs=out_spec,
        out_shape=out_shape,
    )(*args)


def kernel(x, edge_index, W1_l, W1_r, b1, W2_l, W2_r, b2, Wfc, bfc):
    e = edge_index.shape[1]
    quantum = NW * CHUNK * 8
    e_pad = ((e + quantum - 1) // quantum) * quantum
    k_per_worker = e_pad // (NW * CHUNK)
    pad = e_pad - e

    src = jnp.concatenate(
        [edge_index[0].astype(jnp.int32), jnp.zeros((pad,), jnp.int32)]
    ).reshape(-1, CHUNK)
    dst = jnp.concatenate(
        [edge_index[1].astype(jnp.int32), jnp.full((pad,), N, jnp.int32)]
    ).reshape(-1, CHUNK)
    zacc = jnp.zeros((ZROWS, D), jnp.float32)
    zcnt = jnp.zeros((ZROWS, LANES), jnp.float32)
    ones = jnp.ones((CHUNK, LANES), jnp.float32)

    agg1 = _build_sc_aggregate(k_per_worker, with_cnt=True)
    agg2 = _build_sc_aggregate(k_per_worker, with_cnt=False)

    S1, C = agg1(x, src, dst, zacc, zcnt, ones)
    h1 = _tc_layer(S1, C, x, W1_l, W1_r, b1)
    (S2,) = agg2(h1, src, dst, zacc)
    out = _tc_layer(S2, C, h1, W2_l, W2_r, b2, Wfc=Wfc, bfc=bfc)
    return out[:, 0]


# speedup vs baseline: 3.5711x; 1.0022x over previous
import functools

import jax
import jax.numpy as jnp
from jax import lax
from jax.experimental import pallas as pl
from jax.experimental.pallas import tpu as pltpu
from jax.experimental.pallas import tpu_sc as plsc

N = 10000
D = 128
NC, NS = 2, 16
NW = NC * NS
LANES = 16
CHUNK = 64
NB = 4
GROUP = 16
ACC_ROWS = 10112
ZROWS = ACC_ROWS // NS


def _build_sc_aggregate(k_per_worker: int, with_cnt: bool):
    mesh = plsc.VectorSubcoreMesh(
        core_axis_name="c", subcore_axis_name="s", num_cores=NC, num_subcores=NS
    )
    K = k_per_worker
    assert K % NB == 0 and K % GROUP == 0
    ngroups = K // GROUP
    out_type = [jax.ShapeDtypeStruct((NC, ACC_ROWS, D), jnp.float32)]
    scratch = [
        pltpu.VMEM((2 * GROUP, CHUNK), jnp.int32),
        pltpu.VMEM((2 * GROUP, CHUNK), jnp.int32),
    ]
    scratch += [pltpu.VMEM((CHUNK, D), jnp.float32) for _ in range(NB)]
    scratch += [
        pltpu.VMEM_SHARED((ACC_ROWS, D), jnp.float32),
    ]
    scratch += [pltpu.SemaphoreType.DMA for _ in range(2 * NB + 1)]
    if with_cnt:
        out_type.append(jax.ShapeDtypeStruct((NC, ACC_ROWS, LANES), jnp.float32))
        scratch += [
            pltpu.VMEM((CHUNK, LANES), jnp.float32),
            pltpu.VMEM_SHARED((ACC_ROWS, LANES), jnp.float32),
        ]

    def body(*refs):
        if with_cnt:
            (feats, src_h, dst_h, zacc_h, zcnt_h, ones_h,
             out_sum, out_cnt, src_v, dst_v, *rest) = refs
            rows = rest[:NB]
            acc = rest[NB]
            sem_g = rest[NB + 1:NB + 1 + NB]
            sem_s = rest[NB + 1 + NB:NB + 1 + 2 * NB]
            sem_i = rest[NB + 1 + 2 * NB]
            ones_v, cnt_acc = rest[NB + 2 + 2 * NB:]
        else:
            (feats, src_h, dst_h, zacc_h,
             out_sum, src_v, dst_v, *rest) = refs
            rows = rest[:NB]
            acc = rest[NB]
            sem_g = rest[NB + 1:NB + 1 + NB]
            sem_s = rest[NB + 1 + NB:NB + 1 + 2 * NB]
            sem_i = rest[NB + 1 + 2 * NB]
        cid = lax.axis_index("c")
        sid = lax.axis_index("s")
        wid = cid * NS + sid
        ibase = wid * K

        pltpu.sync_copy(zacc_h, acc.at[pl.ds(sid * ZROWS, ZROWS)])
        if with_cnt:
            pltpu.sync_copy(zcnt_h, cnt_acc.at[pl.ds(sid * ZROWS, ZROWS)])
            pltpu.sync_copy(ones_h, ones_v)
        pltpu.sync_copy(src_h.at[pl.ds(ibase, GROUP)], src_v.at[pl.ds(0, GROUP)])
        pltpu.sync_copy(dst_h.at[pl.ds(ibase, GROUP)], dst_v.at[pl.ds(0, GROUP)])
        plsc.subcore_barrier()

        def irow(j):
            return j % (2 * GROUP)

        def gather_desc(j, b):
            return pltpu.make_async_copy(
                feats.at[src_v.at[irow(j)]], rows[b], sem_g[b])

        def scat_start(j, b):
            r = irow(j)
            pltpu.async_copy(rows[b], acc.at[dst_v.at[r]], sem_s[b], add=True)
            if with_cnt:
                pltpu.async_copy(ones_v, cnt_acc.at[dst_v.at[r]], sem_s[b],
                                 add=True)

        def scat_wait(j, b):
            r = irow(j)
            pltpu.make_async_copy(rows[b], acc.at[dst_v.at[r]], sem_s[b]).wait()
            if with_cnt:
                pltpu.make_async_copy(ones_v, cnt_acc.at[dst_v.at[r]],
                                      sem_s[b]).wait()

        def idx_descs(g):
            h = g % 2
            base = ibase + g * GROUP
            return (
                pltpu.make_async_copy(src_h.at[pl.ds(base, GROUP)],
                                      src_v.at[pl.ds(h * GROUP, GROUP)], sem_i),
                pltpu.make_async_copy(dst_h.at[pl.ds(base, GROUP)],
                                      dst_v.at[pl.ds(h * GROUP, GROUP)], sem_i),
            )

        gather_desc(0, 0).start()
        gather_desc(1, 1).start()

        def slot(j, b):
            gather_desc(j, b).wait()

            @pl.when(j >= 1)
            def _():
                scat_wait(j - 1, (b + NB - 1) % NB)

            scat_start(j, b)

            jn = j + 2
            bn = (b + 2) % NB

            @pl.when(jn < K)
            def _():
                @pl.when(jn % GROUP == 0)
                def _():
                    a, bdesc = idx_descs(jn // GROUP)
                    a.wait()
                    bdesc.wait()

                gather_desc(jn, bn).start()

            @pl.when(j % GROUP == 1)
            def _():
                g1 = j // GROUP + 1

                @pl.when(g1 < ngroups)
                def _():
                    a, bdesc = idx_descs(g1)
                    a.start()
                    bdesc.start()

        def step(q, carry):
            j0 = NB * q
            for b in range(NB):
                slot(j0 + b, b)
            return carry

        lax.fori_loop(0, K // NB, step, 0)
        scat_wait(K - 1, (K - 1) % NB)
        plsc.subcore_barrier()

        pltpu.sync_copy(
            acc.at[pl.ds(sid * ZROWS, ZROWS)],
            out_sum.at[cid, pl.ds(sid * ZROWS, ZROWS)],
        )
        if with_cnt:
            pltpu.sync_copy(
                cnt_acc.at[pl.ds(sid * ZROWS, ZROWS)],
                out_cnt.at[cid, pl.ds(sid * ZROWS, ZROWS)],
            )

    return pl.kernel(
        body, out_type=out_type, mesh=mesh, scratch_types=scratch,
        compiler_params=pltpu.CompilerParams(use_tc_tiling_on_sc=False),
    )


def _tc_layer(S, C, xin, W_l, W_r, b, Wfc=None, bfc=None):
    BR = 1000
    final = Wfc is not None

    def body(*refs):
        if final:
            S_r, C_r, x_r, Wl_r, Wr_r, b_r, Wfc_r, bfc_r, o_r = refs
        else:
            S_r, C_r, x_r, Wl_r, Wr_r, b_r, o_r = refs
        s = S_r[0] + S_r[1]
        cnt = C_r[0, :, :1] + C_r[1, :, :1]
        aggr = s / jnp.maximum(cnt, 1.0)
        h = (jnp.dot(aggr, Wl_r[...], preferred_element_type=jnp.float32)
             + jnp.dot(x_r[...], Wr_r[...], preferred_element_type=jnp.float32)
             + b_r[...])
        h = jnp.maximum(h, 0.0)
        if final:
            o_r[...] = (jnp.dot(h, Wfc_r[...], preferred_element_type=jnp.float32)
                        + bfc_r[...])
        else:
            o_r[...] = h

    in_specs = [
        pl.BlockSpec((NC, BR, D), lambda i: (0, i, 0)),
        pl.BlockSpec((NC, BR, LANES), lambda i: (0, i, 0)),
        pl.BlockSpec((BR, D), lambda i: (i, 0)),
        pl.BlockSpec((D, D), lambda i: (0, 0)),
        pl.BlockSpec((D, D), lambda i: (0, 0)),
        pl.BlockSpec((1, D), lambda i: (0, 0)),
    ]
    args = [S, C, xin, W_l, W_r, b.reshape(1, D)]
    if final:
        in_specs += [pl.BlockSpec((D, 1), lambda i: (0, 0)),
                     pl.BlockSpec((1, 1), lambda i: (0, 0))]
        args += [Wfc, bfc.reshape(1, 1)]
        out_spec = pl.BlockSpec((BR, 1), lambda i: (i, 0))
        out_shape = jax.ShapeDtypeStruct((N, 1), jnp.float32)
    else:
        out_spec = pl.BlockSpec((BR, D), lambda i: (i, 0))
        out_shape = jax.ShapeDtypeStruct((N, D), jnp.float32)

    return pl.pallas_call(
        body, grid=(N // BR,), in_specs=in_specs, out_specs=out_spec,
        out_shape=out_shape,
    )(*args)


def kernel(x, edge_index, W1_l, W1_r, b1, W2_l, W2_r, b2, Wfc, bfc):
    e = edge_index.shape[1]
    quantum = NW * CHUNK * 8
    e_pad = ((e + quantum - 1) // quantum) * quantum
    k_per_worker = e_pad // (NW * CHUNK)
    pad = e_pad - e

    src = jnp.concatenate(
        [edge_index[0].astype(jnp.int32), jnp.zeros((pad,), jnp.int32)]
    ).reshape(-1, CHUNK)
    dst = jnp.concatenate(
        [edge_index[1].astype(jnp.int32), jnp.full((pad,), N, jnp.int32)]
    ).reshape(-1, CHUNK)
    zacc = jnp.zeros((ZROWS, D), jnp.float32)
    zcnt = jnp.zeros((ZROWS, LANES), jnp.float32)
    ones = jnp.ones((CHUNK, LANES), jnp.float32)

    agg1 = _build_sc_aggregate(k_per_worker, with_cnt=True)
    agg2 = _build_sc_aggregate(k_per_worker, with_cnt=False)

    S1, C = agg1(x, src, dst, zacc, zcnt, ones)
    h1 = _tc_layer(S1, C, x, W1_l, W1_r, b1)
    (S2,) = agg2(h1, src, dst, zacc)
    out = _tc_layer(S2, C, h1, W2_l, W2_r, b2, Wfc=Wfc, bfc=bfc)
    return out[:, 0]


# speedup vs baseline: 6.5392x; 1.8311x over previous
import jax
import jax.numpy as jnp
from jax import lax
from jax.experimental import pallas as pl
from jax.experimental.pallas import tpu as pltpu
from jax.experimental.pallas import tpu_sc as plsc

N = 10000
D = 128
NC, NS = 2, 16
NW = NC * NS
LANES = 16
CHUNK = 128
NB = 4
GROUP = 16
ACC_ROWS = 10112
ZROWS = ACC_ROWS // NS


def _build_sc_aggregate(k_per_worker: int, with_cnt: bool):
    K = k_per_worker
    assert K % NB == 0 and K % GROUP == 0
    ngroups = K // GROUP
    mesh = plsc.VectorSubcoreMesh(
        core_axis_name="c", subcore_axis_name="s", num_cores=NC, num_subcores=NS
    )
    out_type = [jax.ShapeDtypeStruct((NC, ACC_ROWS, D), jnp.bfloat16)]
    scratch = [
        pltpu.VMEM((2 * GROUP, CHUNK), jnp.int32),
        pltpu.VMEM((2 * GROUP, CHUNK), jnp.int32),
    ]
    scratch += [pltpu.VMEM((CHUNK, D), jnp.bfloat16) for _ in range(NB)]
    scratch += [
        pltpu.VMEM_SHARED((ACC_ROWS, D), jnp.bfloat16),
    ]
    scratch += [pltpu.SemaphoreType.DMA for _ in range(2 * NB + 1)]
    if with_cnt:
        out_type.append(jax.ShapeDtypeStruct((NC, ACC_ROWS, LANES), jnp.bfloat16))
        scratch += [
            pltpu.VMEM((CHUNK, LANES), jnp.bfloat16),
            pltpu.VMEM_SHARED((ACC_ROWS, LANES), jnp.bfloat16),
        ]

    def body(*refs):
        if with_cnt:
            (feats, src_h, dst_h, zacc_h, zcnt_h, ones_h,
             out_sum, out_cnt, src_v, dst_v, *rest) = refs
        else:
            (feats, src_h, dst_h, zacc_h,
             out_sum, src_v, dst_v, *rest) = refs
        rows = rest[:NB]
        acc = rest[NB]
        sem_g = rest[NB + 1:NB + 1 + NB]
        sem_s = rest[NB + 1 + NB:NB + 1 + 2 * NB]
        sem_i = rest[NB + 1 + 2 * NB]
        if with_cnt:
            ones_v, cnt_acc = rest[NB + 2 + 2 * NB:]
        cid = lax.axis_index("c")
        sid = lax.axis_index("s")
        wid = cid * NS + sid
        ibase = wid * K

        pltpu.sync_copy(zacc_h, acc.at[pl.ds(sid * ZROWS, ZROWS)])
        if with_cnt:
            pltpu.sync_copy(zcnt_h, cnt_acc.at[pl.ds(sid * ZROWS, ZROWS)])
            pltpu.sync_copy(ones_h, ones_v)
        pltpu.sync_copy(src_h.at[pl.ds(ibase, GROUP)], src_v.at[pl.ds(0, GROUP)])
        pltpu.sync_copy(dst_h.at[pl.ds(ibase, GROUP)], dst_v.at[pl.ds(0, GROUP)])
        plsc.subcore_barrier()

        def irow(j):
            return j % (2 * GROUP)

        def gather_desc(j, b):
            return pltpu.make_async_copy(
                feats.at[src_v.at[irow(j)]], rows[b], sem_g[b])

        def scat_start(j, b):
            r = irow(j)
            pltpu.async_copy(rows[b], acc.at[dst_v.at[r]], sem_s[b], add=True)
            if with_cnt:
                pltpu.async_copy(ones_v, cnt_acc.at[dst_v.at[r]], sem_s[b],
                                 add=True)

        def scat_wait(j, b):
            r = irow(j)
            pltpu.make_async_copy(rows[b], acc.at[dst_v.at[r]], sem_s[b]).wait()
            if with_cnt:
                pltpu.make_async_copy(ones_v, cnt_acc.at[dst_v.at[r]],
                                      sem_s[b]).wait()

        def idx_descs(g):
            h = g % 2
            base = ibase + g * GROUP
            return (
                pltpu.make_async_copy(src_h.at[pl.ds(base, GROUP)],
                                      src_v.at[pl.ds(h * GROUP, GROUP)], sem_i),
                pltpu.make_async_copy(dst_h.at[pl.ds(base, GROUP)],
                                      dst_v.at[pl.ds(h * GROUP, GROUP)], sem_i),
            )

        gather_desc(0, 0).start()
        gather_desc(1, 1).start()

        def slot(j, b):
            gather_desc(j, b).wait()

            @pl.when(j >= 1)
            def _():
                scat_wait(j - 1, (b + NB - 1) % NB)

            scat_start(j, b)

            jn = j + 2
            bn = (b + 2) % NB

            @pl.when(jn < K)
            def _():
                @pl.when(jn % GROUP == 0)
                def _():
                    a, bdesc = idx_descs(jn // GROUP)
                    a.wait()
                    bdesc.wait()

                gather_desc(jn, bn).start()

            @pl.when(j % GROUP == 1)
            def _():
                g1 = j // GROUP + 1

                @pl.when(g1 < ngroups)
                def _():
                    a, bdesc = idx_descs(g1)
                    a.start()
                    bdesc.start()

        def step(q, carry):
            j0 = NB * q
            for b in range(NB):
                slot(j0 + b, b)
            return carry

        lax.fori_loop(0, K // NB, step, 0)
        scat_wait(K - 1, (K - 1) % NB)
        plsc.subcore_barrier()

        pltpu.sync_copy(
            acc.at[pl.ds(sid * ZROWS, ZROWS)],
            out_sum.at[cid, pl.ds(sid * ZROWS, ZROWS)],
        )
        if with_cnt:
            pltpu.sync_copy(
                cnt_acc.at[pl.ds(sid * ZROWS, ZROWS)],
                out_cnt.at[cid, pl.ds(sid * ZROWS, ZROWS)],
            )

    return pl.kernel(
        body, out_type=out_type, mesh=mesh, scratch_types=scratch,
        compiler_params=pltpu.CompilerParams(use_tc_tiling_on_sc=False),
    )


def _tc_layer(S, C, xin, W_l, W_r, b, Wfc=None, bfc=None):
    BR = 1000
    final = Wfc is not None

    def body(*refs):
        if final:
            S_r, C_r, x_r, Wl_r, Wr_r, b_r, Wfc_r, bfc_r, o_r = refs
        else:
            S_r, C_r, x_r, Wl_r, Wr_r, b_r, o_r, obf_r = refs
        s = S_r[0].astype(jnp.float32) + S_r[1].astype(jnp.float32)
        cnt = (C_r[0, :, :1].astype(jnp.float32)
               + C_r[1, :, :1].astype(jnp.float32))
        aggr = s / jnp.maximum(cnt, 1.0)
        h = (jnp.dot(aggr, Wl_r[...], preferred_element_type=jnp.float32)
             + jnp.dot(x_r[...], Wr_r[...], preferred_element_type=jnp.float32)
             + b_r[...])
        h = jnp.maximum(h, 0.0)
        if final:
            o_r[...] = (jnp.dot(h, Wfc_r[...], preferred_element_type=jnp.float32)
                        + bfc_r[...])
        else:
            o_r[...] = h
            obf_r[...] = h.astype(jnp.bfloat16)

    in_specs = [
        pl.BlockSpec((NC, BR, D), lambda i: (0, i, 0)),
        pl.BlockSpec((NC, BR, LANES), lambda i: (0, i, 0)),
        pl.BlockSpec((BR, D), lambda i: (i, 0)),
        pl.BlockSpec((D, D), lambda i: (0, 0)),
        pl.BlockSpec((D, D), lambda i: (0, 0)),
        pl.BlockSpec((1, D), lambda i: (0, 0)),
    ]
    args = [S, C, xin, W_l, W_r, b.reshape(1, D)]
    if final:
        in_specs += [pl.BlockSpec((D, 1), lambda i: (0, 0)),
                     pl.BlockSpec((1, 1), lambda i: (0, 0))]
        args += [Wfc, bfc.reshape(1, 1)]
        out_spec = pl.BlockSpec((BR, 1), lambda i: (i, 0))
        out_shape = jax.ShapeDtypeStruct((N, 1), jnp.float32)
    else:
        out_spec = [pl.BlockSpec((BR, D), lambda i: (i, 0)),
                    pl.BlockSpec((BR, D), lambda i: (i, 0))]
        out_shape = [jax.ShapeDtypeStruct((N, D), jnp.float32),
                     jax.ShapeDtypeStruct((N, D), jnp.bfloat16)]

    return pl.pallas_call(
        body, grid=(N // BR,), in_specs=in_specs, out_specs=out_spec,
        out_shape=out_shape,
    )(*args)


def kernel(x, edge_index, W1_l, W1_r, b1, W2_l, W2_r, b2, Wfc, bfc):
    e = edge_index.shape[1]
    quantum = NW * CHUNK * GROUP
    e_pad = ((e + quantum - 1) // quantum) * quantum
    k_per_worker = e_pad // (NW * CHUNK)
    pad = e_pad - e

    src = jnp.concatenate(
        [edge_index[0].astype(jnp.int32), jnp.zeros((pad,), jnp.int32)]
    ).reshape(-1, CHUNK)
    dst = jnp.concatenate(
        [edge_index[1].astype(jnp.int32), jnp.full((pad,), N, jnp.int32)]
    ).reshape(-1, CHUNK)
    zacc = jnp.zeros((ZROWS, D), jnp.bfloat16)
    zcnt = jnp.zeros((ZROWS, LANES), jnp.bfloat16)
    ones = jnp.ones((CHUNK, LANES), jnp.bfloat16)
    x_bf = x.astype(jnp.bfloat16)

    agg1 = _build_sc_aggregate(k_per_worker, with_cnt=True)
    agg2 = _build_sc_aggregate(k_per_worker, with_cnt=False)

    S1, C = agg1(x_bf, src, dst, zacc, zcnt, ones)
    h1, h1_bf = _tc_layer(S1, C, x, W1_l, W1_r, b1)
    (S2,) = agg2(h1_bf, src, dst, zacc)
    out = _tc_layer(S2, C, h1, W2_l, W2_r, b2, Wfc=Wfc, bfc=bfc)
    return out[:, 0]
